# baseline (speedup 1.0000x reference)
import functools

import jax
import jax.numpy as jnp
from jax import lax
from jax.experimental import pallas as pl
from jax.experimental.pallas import tpu as pltpu
from jax.experimental.pallas import tpu_sc as plsc

VOCAB = 1000000
HIDDEN = 64
B_TOTAL = 4096 * 200
NC, NS = 2, 16
NW = NC * NS
B_PER_W = B_TOTAL // NW

STEP = 128
SPC = 4
CHUNK = STEP * SPC
N_CHUNKS = B_PER_W // CHUNK
ROWS_PER_W = B_PER_W // STEP


def _emb_body(x_hbm, table_hbm, out_hbm, idx_all, rows_v, sem_g0, sem_g1,
              sem_s0, sem_s1):
    wid = lax.axis_index("s") * NC + lax.axis_index("c")
    idx_row0 = wid * ROWS_PER_W
    out_base = wid * B_PER_W
    sem_g = (sem_g0, sem_g1)
    sem_s = (sem_s0, sem_s1)

    pltpu.sync_copy(x_hbm.at[pl.ds(idx_row0, ROWS_PER_W)], idx_all)

    def fire_gathers(c, b):
        for j in range(SPC):
            pltpu.async_copy(
                table_hbm.at[idx_all.at[c * SPC + j]],
                rows_v.at[b, pl.ds(j * STEP, STEP)],
                sem_g[b],
            )

    def wait_gathers(b):
        for j in range(SPC):
            pltpu.make_async_copy(
                table_hbm.at[idx_all.at[j]],
                rows_v.at[b, pl.ds(j * STEP, STEP)],
                sem_g[b],
            ).wait()

    def start_store(c, b):
        pltpu.async_copy(
            rows_v.at[b],
            out_hbm.at[pl.ds(out_base + c * CHUNK, CHUNK)],
            sem_s[b],
        )

    def wait_store(b):
        pltpu.make_async_copy(
            rows_v.at[b],
            out_hbm.at[pl.ds(out_base, CHUNK)],
            sem_s[b],
        ).wait()

    def steady(c, b):
        wait_store(1 - b)
        fire_gathers(c + 1, 1 - b)
        wait_gathers(b)
        start_store(c, b)

    fire_gathers(0, 0)
    fire_gathers(1, 1)
    wait_gathers(0)
    start_store(0, 0)

    def chunk_pair(i, _):
        steady(2 * i + 1, 1)
        steady(2 * i + 2, 0)
        return ()

    lax.fori_loop(0, (N_CHUNKS - 2) // 2, chunk_pair, ())

    wait_store(0)
    wait_gathers(1)
    start_store(N_CHUNKS - 1, 1)
    wait_store(1)


_emb = functools.partial(
    pl.kernel,
    mesh=plsc.VectorSubcoreMesh(core_axis_name="c", subcore_axis_name="s"),
    out_type=jax.ShapeDtypeStruct((B_TOTAL, HIDDEN), jnp.float32),
    scratch_types=[
        pltpu.VMEM((ROWS_PER_W, STEP), jnp.int32),
        pltpu.VMEM((2, CHUNK, HIDDEN), jnp.float32),
        pltpu.SemaphoreType.DMA,
        pltpu.SemaphoreType.DMA,
        pltpu.SemaphoreType.DMA,
        pltpu.SemaphoreType.DMA,
    ],
    compiler_params=pltpu.CompilerParams(use_tc_tiling_on_sc=False),
)(_emb_body)


def kernel(x, table):
    x_rows = x.reshape(B_TOTAL // STEP, STEP)
    out = _emb(x_rows, table)
    return out.reshape(x.shape[0], x.shape[1], HIDDEN)
